# baseline (speedup 1.0000x reference)
import functools

import jax
import jax.numpy as jnp
from jax import lax
from jax.experimental import pallas as pl
from jax.experimental.pallas import tpu as pltpu
from jax.experimental.pallas import tpu_sc as plsc

N = 10000
E = 320000
D = 128
ROOT = 2048

NC, NS = 2, 16
NW = NC * NS
C = 128
K = 80
EPAD = NW * K * C
FAST_C = 0
KA = 140
KB = 2 * K - KA

NR0 = 10240
RPT0 = NR0 // NS
DUMP0 = N
NR1 = 2176
RPT1 = NR1 // NS
DUMP1 = ROOT


def _make_seg_sum(nr, rpt, with_cnt, filt=False, ka=KA, kb=KB):
    mesh = plsc.VectorSubcoreMesh(
        core_axis_name="c", subcore_axis_name="s",
        num_cores=NC, num_subcores=NS)
    qr = nr // C

    outs = [jax.ShapeDtypeStruct((NC, nr, D), jnp.float32)]
    scratch = [
        pltpu.VMEM((2, C), jnp.int32),
        pltpu.VMEM((2, C), jnp.int32),
        pltpu.VMEM((2, C, D), jnp.float32),
        pltpu.VMEM_SHARED((nr, D), jnp.float32),
        pltpu.SemaphoreType.DMA,
        pltpu.SemaphoreType.DMA,
        pltpu.SemaphoreType.DMA,
        pltpu.SemaphoreType.DMA,
    ]
    if with_cnt:
        outs.append(jax.ShapeDtypeStruct((NC, qr, C), jnp.float32))
        scratch += [
            pltpu.VMEM((qr, C), jnp.float32),
            pltpu.VMEM((qr,), jnp.int32),
            pltpu.VMEM_SHARED((qr, C), jnp.float32),
        ]
    if filt:
        scratch += [
            pltpu.VMEM((ka * C + 2 * C,), jnp.int32),
            pltpu.VMEM((ka * C + 2 * C,), jnp.int32),
            pltpu.VMEM((ka + 1, C), jnp.int32),
        ]

    @functools.partial(
        pl.kernel, out_type=tuple(outs), mesh=mesh,
        scratch_types=scratch,
        compiler_params=pltpu.CompilerParams(needs_layout_passes=False),
    )
    def seg(table, srcs, dsts, zrows, *rest):
        if with_cnt:
            (out, cnt_out, src_v, dst_v, rows_v, accum, semg0, semg1,
             semis, semid, cnt_v, idr_v, cnt_sh) = rest
        elif filt:
            (out, src_v, dst_v, rows_v, accum, semg0, semg1,
             semis, semid, sfil, dfil, dfil2) = rest
        else:
            (out, src_v, dst_v, rows_v, accum, semg0, semg1,
             semis, semid) = rest
        semg = (semg0, semg1)
        c = lax.axis_index("c")
        s = lax.axis_index("s")
        fast = c == FAST_C
        kc = jnp.where(fast, ka, kb)
        ebase = jnp.where(fast, s * ka, NS * ka + s * kb) * C

        def idx_copies(j, b):
            return (
                pltpu.make_async_copy(
                    srcs.at[pl.ds(ebase + j * C, C)], src_v.at[b], semis),
                pltpu.make_async_copy(
                    dsts.at[pl.ds(ebase + j * C, C)], dst_v.at[b], semid),
            )

        def gather_copy(j, b):
            del j
            return pltpu.make_async_copy(
                table.at[src_v.at[b]], rows_v.at[b], semg[b])

        pltpu.sync_copy(zrows, rows_v.at[0])
        base = s * rpt
        for q in range(rpt // C):
            pltpu.sync_copy(rows_v.at[0], accum.at[pl.ds(base + q * C, C)])
        rem = rpt % C
        if rem:
            pltpu.sync_copy(rows_v.at[0, pl.ds(0, rem)],
                            accum.at[pl.ds(base + (rpt // C) * C, rem)])
        if with_cnt:
            pltpu.sync_copy(zrows.at[pl.ds(0, qr)], cnt_v)
            for q in range(qr // 16):
                idr_v[pl.ds(q * 16, 16)] = (
                    lax.iota(jnp.int32, 16) + q * 16)

            @pl.when(s == 0)
            def _():
                pltpu.sync_copy(zrows.at[pl.ds(0, qr)], cnt_sh)

        plsc.subcore_barrier()

        ones = jnp.ones((16,), jnp.float32)

        def count_chunk(b):
            def cstep(i, carry):
                idx = dst_v[b, pl.ds(i * 16, 16)]
                hi = lax.shift_right_logical(idx, 7)
                lo = lax.bitwise_and(idx, 127)
                plsc.addupdate_scatter(cnt_v, [hi, lo], ones)
                return carry
            lax.fori_loop(0, C // 16, cstep, 0)

        if not filt:
            @pl.when(kc > 0)
            def _():
                for d in idx_copies(0, 0):
                    d.start()
                for d in idx_copies(0, 0):
                    d.wait()
                gather_copy(0, 0).start()
                for d in idx_copies(1, 1):
                    d.start()

            def half(j, b):
                nb = 1 - b
                gather_copy(j, b).wait()

                @pl.when(j + 1 < kc)
                def _():
                    for d in idx_copies(j + 1, nb):
                        d.wait()
                    gather_copy(j + 1, nb).start()

                if with_cnt:
                    count_chunk(b)
                pltpu.sync_copy(rows_v.at[b], accum.at[dst_v.at[b]],
                                add=True)

                @pl.when(j + 2 < kc)
                def _():
                    for d in idx_copies(j + 2, b):
                        d.start()

            def round_(r, carry):
                half(2 * r, 0)
                half(2 * r + 1, 1)
                return carry

            lax.fori_loop(0, kc // 2, round_, 0)
        else:
            @pl.when(kc > 0)
            def _():
                for d in idx_copies(0, 0):
                    d.start()

            def fhalf(j, b, n):
                nb = 1 - b
                for d in idx_copies(j, b):
                    d.wait()

                @pl.when(j + 1 < kc)
                def _():
                    for d in idx_copies(j + 1, nb):
                        d.start()

                def cstep(i, n):
                    svec = src_v[b, pl.ds(i * 16, 16)]
                    dvec = dst_v[b, pl.ds(i * 16, 16)]
                    mask = dvec < DUMP1
                    plsc.store_compressed(
                        sfil.at[pl.ds(n, 16)], svec, mask=mask)
                    plsc.store_compressed(
                        dfil.at[pl.ds(n, 16)], dvec, mask=mask)
                    return n + plsc.all_reduce_population_count(mask)[0]

                return lax.fori_loop(0, C // 16, cstep, n)

            def fround(r, n):
                n = fhalf(2 * r, 0, n)
                n = fhalf(2 * r + 1, 1, n)
                return n

            n = lax.fori_loop(0, kc // 2, fround, jnp.int32(0))

            zeros16 = jnp.zeros((16,), jnp.int32)
            dump16 = jnp.full((16,), DUMP1, jnp.int32)
            for t in range(C // 16):
                sfil[pl.ds(n + t * 16, 16)] = zeros16
                dfil[pl.ds(n + t * 16, 16)] = dump16
            nch = (n + C - 1) // C

            def rstep(q, carry):
                def ristep(i, carry2):
                    dfil2[q, pl.ds(i * 16, 16)] = (
                        dfil[pl.ds(q * C + i * 16, 16)])
                    return carry2
                return lax.fori_loop(0, C // 16, ristep, carry)

            lax.fori_loop(0, nch, rstep, 0)

            def gather_f(j, b):
                return pltpu.make_async_copy(
                    table.at[sfil.at[pl.ds(j * C, C)]], rows_v.at[b],
                    semg[b])

            @pl.when(nch > 0)
            def _():
                gather_f(0, 0).start()

            def mhalf(j, b):
                nb = 1 - b
                gather_f(j, b).wait()

                @pl.when(j + 1 < nch)
                def _():
                    gather_f(j + 1, nb).start()

                pltpu.sync_copy(rows_v.at[b], accum.at[dfil2.at[j]],
                                add=True)

            def mround(r, carry):
                mhalf(2 * r, 0)

                @pl.when(2 * r + 1 < nch)
                def _():
                    mhalf(2 * r + 1, 1)
                return carry

            lax.fori_loop(0, (nch + 1) // 2, mround, 0)
        if with_cnt:
            pltpu.sync_copy(cnt_v, cnt_sh.at[idr_v], add=True)
        plsc.subcore_barrier()

        pltpu.sync_copy(accum.at[pl.ds(base, rpt)],
                        out.at[c, pl.ds(base, rpt)])
        if with_cnt:
            @pl.when(s < qr // 8)
            def _():
                pltpu.sync_copy(cnt_sh.at[pl.ds(s * 8, 8)],
                                cnt_out.at[c, pl.ds(s * 8, 8)])

    return seg


_make_seg_sum = functools.lru_cache(maxsize=None)(_make_seg_sum)


def _make_dense(rows, blk, relu):
    nb = rows // blk

    def body(s0, s1, c0, c1, xr, wl, bl, wr, o):
        cnt = jnp.maximum(c0[...] + c1[...], 1.0)
        mean = (s0[...] + s1[...]) / cnt
        h = jnp.dot(mean, wl[...], preferred_element_type=jnp.float32)
        h = h + jnp.dot(xr[...], wr[...], preferred_element_type=jnp.float32)
        h = h + bl[...]
        o[...] = jnp.maximum(h, 0.0) if relu else h

    rb = pl.BlockSpec((blk, D), lambda i: (i, 0))
    cb = pl.BlockSpec((blk, 1), lambda i: (i, 0))
    wb = pl.BlockSpec((D, D), lambda i: (0, 0))
    bb = pl.BlockSpec((1, D), lambda i: (0, 0))
    return pl.pallas_call(
        body,
        grid=(nb,),
        in_specs=[rb, rb, cb, cb, rb, wb, bb, wb],
        out_specs=rb,
        out_shape=jax.ShapeDtypeStruct((rows, D), jnp.float32),
    )


_dense0 = _make_dense(N, 1000, True)
_dense1 = _make_dense(ROOT, 1024, False)


def kernel(x, nbrs, num_root, Wl0, bl0, Wr0, Wl1, bl1, Wr1):
    src = nbrs[0]
    dst = nbrs[1]
    pad = EPAD - E

    srcp = jnp.concatenate([src, jnp.zeros((pad,), jnp.int32)])
    dst0p = jnp.concatenate([dst, jnp.full((pad,), DUMP0, jnp.int32)])

    z = jnp.zeros((C, D), jnp.float32)

    p0, cnt = _make_seg_sum(NR0, RPT0, True)(x, srcp, dst0p, z)
    c0 = cnt[0].reshape(NR0, 1)[:N]
    c1 = cnt[1].reshape(NR0, 1)[:N]
    h = _dense0(p0[0, :N], p0[1, :N], c0, c1,
                x, Wl0.T, bl0.reshape(1, D), Wr0.T)

    (p1,) = _make_seg_sum(NR1, RPT1, False, filt=True,
                          ka=104, kb=56)(h, srcp, dst0p, z)
    out = _dense1(p1[0, :ROOT], p1[1, :ROOT], c0[:ROOT], c1[:ROOT],
                  h[:ROOT], Wl1.T, bl1.reshape(1, D), Wr1.T)
    return out


# speedup vs baseline: 1.0049x; 1.0049x over previous
import functools

import jax
import jax.numpy as jnp
from jax import lax
from jax.experimental import pallas as pl
from jax.experimental.pallas import tpu as pltpu
from jax.experimental.pallas import tpu_sc as plsc

N = 10000
E = 320000
D = 128
ROOT = 2048

NC, NS = 2, 16
NW = NC * NS
C = 128
K = 80
EPAD = NW * K * C
FAST_C = 0
KA = 144
KB = 2 * K - KA

NR0 = 10240
RPT0 = NR0 // NS
DUMP0 = N
NR1 = 2176
RPT1 = NR1 // NS
DUMP1 = ROOT


def _make_seg_sum(nr, rpt, with_cnt, filt=False, ka=KA, kb=KB):
    mesh = plsc.VectorSubcoreMesh(
        core_axis_name="c", subcore_axis_name="s",
        num_cores=NC, num_subcores=NS)
    qr = nr // C

    outs = [jax.ShapeDtypeStruct((NC, nr, D), jnp.float32)]
    scratch = [
        pltpu.VMEM((2, C), jnp.int32),
        pltpu.VMEM((2, C), jnp.int32),
        pltpu.VMEM((2, C, D), jnp.float32),
        pltpu.VMEM_SHARED((nr, D), jnp.float32),
        pltpu.SemaphoreType.DMA,
        pltpu.SemaphoreType.DMA,
        pltpu.SemaphoreType.DMA,
        pltpu.SemaphoreType.DMA,
    ]
    if with_cnt:
        outs.append(jax.ShapeDtypeStruct((NC, qr, C), jnp.float32))
        scratch += [
            pltpu.VMEM((qr, C), jnp.float32),
            pltpu.VMEM((qr,), jnp.int32),
            pltpu.VMEM_SHARED((qr, C), jnp.float32),
        ]
    if filt:
        scratch += [
            pltpu.VMEM((ka * C + 2 * C,), jnp.int32),
            pltpu.VMEM((ka * C + 2 * C,), jnp.int32),
            pltpu.VMEM((ka + 1, C), jnp.int32),
        ]

    @functools.partial(
        pl.kernel, out_type=tuple(outs), mesh=mesh,
        scratch_types=scratch,
        compiler_params=pltpu.CompilerParams(needs_layout_passes=False),
    )
    def seg(table, srcs, dsts, zrows, *rest):
        if with_cnt:
            (out, cnt_out, src_v, dst_v, rows_v, accum, semg0, semg1,
             semis, semid, cnt_v, idr_v, cnt_sh) = rest
        elif filt:
            (out, src_v, dst_v, rows_v, accum, semg0, semg1,
             semis, semid, sfil, dfil, dfil2) = rest
        else:
            (out, src_v, dst_v, rows_v, accum, semg0, semg1,
             semis, semid) = rest
        semg = (semg0, semg1)
        c = lax.axis_index("c")
        s = lax.axis_index("s")
        fast = c == FAST_C
        kc = jnp.where(fast, ka, kb)
        ebase = jnp.where(fast, s * ka, NS * ka + s * kb) * C

        def idx_copies(j, b):
            return (
                pltpu.make_async_copy(
                    srcs.at[pl.ds(ebase + j * C, C)], src_v.at[b], semis),
                pltpu.make_async_copy(
                    dsts.at[pl.ds(ebase + j * C, C)], dst_v.at[b], semid),
            )

        def gather_copy(j, b):
            del j
            return pltpu.make_async_copy(
                table.at[src_v.at[b]], rows_v.at[b], semg[b])

        pltpu.sync_copy(zrows, rows_v.at[0])
        base = s * rpt
        for q in range(rpt // C):
            pltpu.sync_copy(rows_v.at[0], accum.at[pl.ds(base + q * C, C)])
        rem = rpt % C
        if rem:
            pltpu.sync_copy(rows_v.at[0, pl.ds(0, rem)],
                            accum.at[pl.ds(base + (rpt // C) * C, rem)])
        if with_cnt:
            pltpu.sync_copy(zrows.at[pl.ds(0, qr)], cnt_v)
            for q in range(qr // 16):
                idr_v[pl.ds(q * 16, 16)] = (
                    lax.iota(jnp.int32, 16) + q * 16)

            @pl.when(s == 0)
            def _():
                pltpu.sync_copy(zrows.at[pl.ds(0, qr)], cnt_sh)

        plsc.subcore_barrier()

        ones = jnp.ones((16,), jnp.float32)

        def count_chunk(b):
            def cstep(i, carry):
                idx = dst_v[b, pl.ds(i * 16, 16)]
                hi = lax.shift_right_logical(idx, 7)
                lo = lax.bitwise_and(idx, 127)
                plsc.addupdate_scatter(cnt_v, [hi, lo], ones)
                return carry
            lax.fori_loop(0, C // 16, cstep, 0)

        if not filt:
            @pl.when(kc > 0)
            def _():
                for d in idx_copies(0, 0):
                    d.start()
                for d in idx_copies(0, 0):
                    d.wait()
                gather_copy(0, 0).start()
                for d in idx_copies(1, 1):
                    d.start()

            def half(j, b):
                nb = 1 - b
                gather_copy(j, b).wait()

                @pl.when(j + 1 < kc)
                def _():
                    for d in idx_copies(j + 1, nb):
                        d.wait()
                    gather_copy(j + 1, nb).start()

                if with_cnt:
                    count_chunk(b)
                pltpu.sync_copy(rows_v.at[b], accum.at[dst_v.at[b]],
                                add=True)

                @pl.when(j + 2 < kc)
                def _():
                    for d in idx_copies(j + 2, b):
                        d.start()

            def round_(r, carry):
                half(2 * r, 0)
                half(2 * r + 1, 1)
                return carry

            lax.fori_loop(0, kc // 2, round_, 0)
        else:
            @pl.when(kc > 0)
            def _():
                for d in idx_copies(0, 0):
                    d.start()

            def fhalf(j, b, n):
                nb = 1 - b
                for d in idx_copies(j, b):
                    d.wait()

                @pl.when(j + 1 < kc)
                def _():
                    for d in idx_copies(j + 1, nb):
                        d.start()

                def cstep(i, n):
                    svec = src_v[b, pl.ds(i * 16, 16)]
                    dvec = dst_v[b, pl.ds(i * 16, 16)]
                    mask = dvec < DUMP1
                    plsc.store_compressed(
                        sfil.at[pl.ds(n, 16)], svec, mask=mask)
                    plsc.store_compressed(
                        dfil.at[pl.ds(n, 16)], dvec, mask=mask)
                    return n + plsc.all_reduce_population_count(mask)[0]

                return lax.fori_loop(0, C // 16, cstep, n)

            def fround(r, n):
                n = fhalf(2 * r, 0, n)
                n = fhalf(2 * r + 1, 1, n)
                return n

            n = lax.fori_loop(0, kc // 2, fround, jnp.int32(0))

            zeros16 = jnp.zeros((16,), jnp.int32)
            dump16 = jnp.full((16,), DUMP1, jnp.int32)
            for t in range(C // 16):
                sfil[pl.ds(n + t * 16, 16)] = zeros16
                dfil[pl.ds(n + t * 16, 16)] = dump16
            nch = (n + C - 1) // C

            def rstep(q, carry):
                def ristep(i, carry2):
                    dfil2[q, pl.ds(i * 16, 16)] = (
                        dfil[pl.ds(q * C + i * 16, 16)])
                    return carry2
                return lax.fori_loop(0, C // 16, ristep, carry)

            lax.fori_loop(0, nch, rstep, 0)

            def gather_f(j, b):
                return pltpu.make_async_copy(
                    table.at[sfil.at[pl.ds(j * C, C)]], rows_v.at[b],
                    semg[b])

            @pl.when(nch > 0)
            def _():
                gather_f(0, 0).start()

            def mhalf(j, b):
                nb = 1 - b
                gather_f(j, b).wait()

                @pl.when(j + 1 < nch)
                def _():
                    gather_f(j + 1, nb).start()

                pltpu.sync_copy(rows_v.at[b], accum.at[dfil2.at[j]],
                                add=True)

            def mround(r, carry):
                mhalf(2 * r, 0)

                @pl.when(2 * r + 1 < nch)
                def _():
                    mhalf(2 * r + 1, 1)
                return carry

            lax.fori_loop(0, (nch + 1) // 2, mround, 0)
        if with_cnt:
            pltpu.sync_copy(cnt_v, cnt_sh.at[idr_v], add=True)
        plsc.subcore_barrier()

        pltpu.sync_copy(accum.at[pl.ds(base, rpt)],
                        out.at[c, pl.ds(base, rpt)])
        if with_cnt:
            @pl.when(s < qr // 8)
            def _():
                pltpu.sync_copy(cnt_sh.at[pl.ds(s * 8, 8)],
                                cnt_out.at[c, pl.ds(s * 8, 8)])

    return seg


_make_seg_sum = functools.lru_cache(maxsize=None)(_make_seg_sum)


def _make_dense(rows, blk, relu):
    nb = rows // blk

    def body(s0, s1, c0, c1, xr, wl, bl, wr, o):
        cnt = jnp.maximum(c0[...] + c1[...], 1.0)
        mean = (s0[...] + s1[...]) / cnt
        h = jnp.dot(mean, wl[...], preferred_element_type=jnp.float32)
        h = h + jnp.dot(xr[...], wr[...], preferred_element_type=jnp.float32)
        h = h + bl[...]
        o[...] = jnp.maximum(h, 0.0) if relu else h

    rb = pl.BlockSpec((blk, D), lambda i: (i, 0))
    cb = pl.BlockSpec((blk, 1), lambda i: (i, 0))
    wb = pl.BlockSpec((D, D), lambda i: (0, 0))
    bb = pl.BlockSpec((1, D), lambda i: (0, 0))
    return pl.pallas_call(
        body,
        grid=(nb,),
        in_specs=[rb, rb, cb, cb, rb, wb, bb, wb],
        out_specs=rb,
        out_shape=jax.ShapeDtypeStruct((rows, D), jnp.float32),
    )


_dense0 = _make_dense(N, 1000, True)
_dense1 = _make_dense(ROOT, 1024, False)


def kernel(x, nbrs, num_root, Wl0, bl0, Wr0, Wl1, bl1, Wr1):
    src = nbrs[0]
    dst = nbrs[1]
    pad = EPAD - E

    srcp = jnp.concatenate([src, jnp.zeros((pad,), jnp.int32)])
    dst0p = jnp.concatenate([dst, jnp.full((pad,), DUMP0, jnp.int32)])

    z = jnp.zeros((C, D), jnp.float32)

    p0, cnt = _make_seg_sum(NR0, RPT0, True)(x, srcp, dst0p, z)
    c0 = cnt[0].reshape(NR0, 1)[:N]
    c1 = cnt[1].reshape(NR0, 1)[:N]
    h = _dense0(p0[0, :N], p0[1, :N], c0, c1,
                x, Wl0.T, bl0.reshape(1, D), Wr0.T)

    (p1,) = _make_seg_sum(NR1, RPT1, False, filt=True,
                          ka=112, kb=48)(h, srcp, dst0p, z)
    out = _dense1(p1[0, :ROOT], p1[1, :ROOT], c0[:ROOT], c1[:ROOT],
                  h[:ROOT], Wl1.T, bl1.reshape(1, D), Wr1.T)
    return out


# speedup vs baseline: 1.0403x; 1.0352x over previous
import functools

import jax
import jax.numpy as jnp
from jax import lax
from jax.experimental import pallas as pl
from jax.experimental.pallas import tpu as pltpu
from jax.experimental.pallas import tpu_sc as plsc

N = 10000
E = 320000
D = 128
ROOT = 2048

NC, NS = 2, 16
NW = NC * NS
C = 128
K = 80
EPAD = NW * K * C
FAST_C = 0
KA = 144
KB = 2 * K - KA

NR0 = 10240
RPT0 = NR0 // NS
DUMP0 = N
NR1 = 2176
RPT1 = NR1 // NS
DUMP1 = ROOT


def _make_seg_sum(nr, rpt, with_cnt, filt=False, ka=KA, kb=KB):
    mesh = plsc.VectorSubcoreMesh(
        core_axis_name="c", subcore_axis_name="s",
        num_cores=NC, num_subcores=NS)
    qr = nr // C

    outs = [jax.ShapeDtypeStruct((NC, nr, D), jnp.float32)]
    scratch = [
        pltpu.VMEM((2, C), jnp.int32),
        pltpu.VMEM((2, C), jnp.int32),
        pltpu.VMEM((2, C, D), jnp.float32),
        pltpu.VMEM_SHARED((nr, D), jnp.float32),
        pltpu.SemaphoreType.DMA,
        pltpu.SemaphoreType.DMA,
        pltpu.SemaphoreType.DMA,
        pltpu.SemaphoreType.DMA,
    ]
    if with_cnt:
        outs.append(jax.ShapeDtypeStruct((NC, qr, C), jnp.float32))
        scratch += [
            pltpu.VMEM((qr, C), jnp.float32),
            pltpu.VMEM((qr,), jnp.int32),
            pltpu.VMEM_SHARED((qr, C), jnp.float32),
        ]
    if filt:
        scratch += [
            pltpu.VMEM((ka * C + 2 * C,), jnp.int32),
            pltpu.VMEM((ka * C + 2 * C,), jnp.int32),
            pltpu.VMEM((ka + 1, C), jnp.int32),
        ]

    @functools.partial(
        pl.kernel, out_type=tuple(outs), mesh=mesh,
        scratch_types=scratch,
        compiler_params=pltpu.CompilerParams(needs_layout_passes=False),
    )
    def seg(table, srcs, dsts, zrows, *rest):
        if with_cnt:
            (out, cnt_out, src_v, dst_v, rows_v, accum, semg0, semg1,
             semis, semid, cnt_v, idr_v, cnt_sh) = rest
        elif filt:
            (out, src_v, dst_v, rows_v, accum, semg0, semg1,
             semis, semid, sfil, dfil, dfil2) = rest
        else:
            (out, src_v, dst_v, rows_v, accum, semg0, semg1,
             semis, semid) = rest
        semg = (semg0, semg1)
        c = lax.axis_index("c")
        s = lax.axis_index("s")
        fast = c == FAST_C
        kc = jnp.where(fast, ka, kb)
        ebase = jnp.where(fast, s * ka, NS * ka + s * kb) * C

        def idx_copies(j, b):
            return (
                pltpu.make_async_copy(
                    srcs.at[pl.ds(ebase + j * C, C)], src_v.at[b], semis),
                pltpu.make_async_copy(
                    dsts.at[pl.ds(ebase + j * C, C)], dst_v.at[b], semid),
            )

        def gather_copy(j, b):
            del j
            return pltpu.make_async_copy(
                table.at[src_v.at[b]], rows_v.at[b], semg[b])

        pltpu.sync_copy(zrows, rows_v.at[0])
        base = s * rpt
        for q in range(rpt // C):
            pltpu.sync_copy(rows_v.at[0], accum.at[pl.ds(base + q * C, C)])
        rem = rpt % C
        if rem:
            pltpu.sync_copy(rows_v.at[0, pl.ds(0, rem)],
                            accum.at[pl.ds(base + (rpt // C) * C, rem)])
        if with_cnt:
            pltpu.sync_copy(zrows.at[pl.ds(0, qr)], cnt_v)
            for q in range(qr // 16):
                idr_v[pl.ds(q * 16, 16)] = (
                    lax.iota(jnp.int32, 16) + q * 16)

            @pl.when(s == 0)
            def _():
                pltpu.sync_copy(zrows.at[pl.ds(0, qr)], cnt_sh)

        plsc.subcore_barrier()

        ones = jnp.ones((16,), jnp.float32)

        def count_chunk(b):
            def cstep(i, carry):
                idx = dst_v[b, pl.ds(i * 16, 16)]
                hi = lax.shift_right_logical(idx, 7)
                lo = lax.bitwise_and(idx, 127)
                plsc.addupdate_scatter(cnt_v, [hi, lo], ones)
                return carry
            lax.fori_loop(0, C // 16, cstep, 0)

        if not filt:
            @pl.when(kc > 0)
            def _():
                for d in idx_copies(0, 0):
                    d.start()
                for d in idx_copies(0, 0):
                    d.wait()
                gather_copy(0, 0).start()
                for d in idx_copies(1, 1):
                    d.start()

            def half(j, b):
                nb = 1 - b
                gather_copy(j, b).wait()

                @pl.when(j + 1 < kc)
                def _():
                    for d in idx_copies(j + 1, nb):
                        d.wait()
                    gather_copy(j + 1, nb).start()

                if with_cnt:
                    count_chunk(b)
                pltpu.sync_copy(rows_v.at[b], accum.at[dst_v.at[b]],
                                add=True)

                @pl.when(j + 2 < kc)
                def _():
                    for d in idx_copies(j + 2, b):
                        d.start()

            def round_(r, carry):
                half(2 * r, 0)
                half(2 * r + 1, 1)
                return carry

            lax.fori_loop(0, kc // 2, round_, 0)
        else:
            @pl.when(kc > 0)
            def _():
                for d in idx_copies(0, 0):
                    d.start()

            def fhalf(j, b, n):
                nb = 1 - b
                for d in idx_copies(j, b):
                    d.wait()

                @pl.when(j + 1 < kc)
                def _():
                    for d in idx_copies(j + 1, nb):
                        d.start()

                def cstep(i, n):
                    svec = src_v[b, pl.ds(i * 16, 16)]
                    dvec = dst_v[b, pl.ds(i * 16, 16)]
                    mask = dvec < DUMP1
                    plsc.store_compressed(
                        sfil.at[pl.ds(n, 16)], svec, mask=mask)
                    plsc.store_compressed(
                        dfil.at[pl.ds(n, 16)], dvec, mask=mask)
                    return n + plsc.all_reduce_population_count(mask)[0]

                return lax.fori_loop(0, C // 16, cstep, n)

            def fround(r, n):
                n = fhalf(2 * r, 0, n)
                n = fhalf(2 * r + 1, 1, n)
                return n

            n = lax.fori_loop(0, kc // 2, fround, jnp.int32(0))

            zeros16 = jnp.zeros((16,), jnp.int32)
            dump16 = jnp.full((16,), DUMP1, jnp.int32)
            for t in range(C // 16):
                sfil[pl.ds(n + t * 16, 16)] = zeros16
                dfil[pl.ds(n + t * 16, 16)] = dump16
            nch = (n + C - 1) // C

            def rstep(q, carry):
                def ristep(i, carry2):
                    dfil2[q, pl.ds(i * 16, 16)] = (
                        dfil[pl.ds(q * C + i * 16, 16)])
                    return carry2
                return lax.fori_loop(0, C // 16, ristep, carry)

            lax.fori_loop(0, nch, rstep, 0)

            def gather_f(j, b):
                return pltpu.make_async_copy(
                    table.at[sfil.at[pl.ds(j * C, C)]], rows_v.at[b],
                    semg[b])

            @pl.when(nch > 0)
            def _():
                gather_f(0, 0).start()

            def mhalf(j, b):
                nb = 1 - b
                gather_f(j, b).wait()

                @pl.when(j + 1 < nch)
                def _():
                    gather_f(j + 1, nb).start()

                pltpu.sync_copy(rows_v.at[b], accum.at[dfil2.at[j]],
                                add=True)

            def mround(r, carry):
                mhalf(2 * r, 0)

                @pl.when(2 * r + 1 < nch)
                def _():
                    mhalf(2 * r + 1, 1)
                return carry

            lax.fori_loop(0, (nch + 1) // 2, mround, 0)
        if with_cnt:
            pltpu.sync_copy(cnt_v, cnt_sh.at[idr_v], add=True)
        plsc.subcore_barrier()

        pltpu.sync_copy(accum.at[pl.ds(base, rpt)],
                        out.at[c, pl.ds(base, rpt)])
        if with_cnt:
            @pl.when(s < qr // 8)
            def _():
                pltpu.sync_copy(cnt_sh.at[pl.ds(s * 8, 8)],
                                cnt_out.at[c, pl.ds(s * 8, 8)])

    return seg


_make_seg_sum = functools.lru_cache(maxsize=None)(_make_seg_sum)


def _make_dense(rows, blk, relu):
    nb = rows // blk

    def body(s0, s1, c0, c1, xr, wl, bl, wr, o):
        cnt = jnp.maximum(c0[...] + c1[...], 1.0)
        mean = (s0[...] + s1[...]) / cnt
        h = jnp.dot(mean, wl[...], preferred_element_type=jnp.float32)
        h = h + jnp.dot(xr[...], wr[...], preferred_element_type=jnp.float32)
        h = h + bl[...]
        o[...] = jnp.maximum(h, 0.0) if relu else h

    rb = pl.BlockSpec((blk, D), lambda i: (i, 0))
    cb = pl.BlockSpec((blk, 1), lambda i: (i, 0))
    wb = pl.BlockSpec((D, D), lambda i: (0, 0))
    bb = pl.BlockSpec((1, D), lambda i: (0, 0))
    return pl.pallas_call(
        body,
        grid=(nb,),
        in_specs=[rb, rb, cb, cb, rb, wb, bb, wb],
        out_specs=rb,
        out_shape=jax.ShapeDtypeStruct((rows, D), jnp.float32),
    )


_dense0 = _make_dense(N, 1000, True)
_dense1 = _make_dense(ROOT, 1024, False)


def kernel(x, nbrs, num_root, Wl0, bl0, Wr0, Wl1, bl1, Wr1):
    src = nbrs[0]
    dst = nbrs[1]
    pad = EPAD - E

    srcp = jnp.concatenate([src, jnp.zeros((pad,), jnp.int32)])
    dst0p = jnp.concatenate([dst, jnp.full((pad,), DUMP0, jnp.int32)])

    z = jnp.zeros((C, D), jnp.float32)

    p0, cnt = _make_seg_sum(NR0, RPT0, True)(x, srcp, dst0p, z)
    c0 = cnt[0].reshape(NR0, 1)[:N]
    c1 = cnt[1].reshape(NR0, 1)[:N]
    h = _dense0(p0[0, :N], p0[1, :N], c0, c1,
                x, Wl0.T, bl0.reshape(1, D), Wr0.T)

    (p1,) = _make_seg_sum(NR1, RPT1, False, filt=True,
                          ka=104, kb=56)(h, srcp, dst0p, z)
    out = _dense1(p1[0, :ROOT], p1[1, :ROOT], c0[:ROOT], c1[:ROOT],
                  h[:ROOT], Wl1.T, bl1.reshape(1, D), Wr1.T)
    return out


# speedup vs baseline: 1.0507x; 1.0100x over previous
import functools

import jax
import jax.numpy as jnp
from jax import lax
from jax.experimental import pallas as pl
from jax.experimental.pallas import tpu as pltpu
from jax.experimental.pallas import tpu_sc as plsc

N = 10000
E = 320000
D = 128
ROOT = 2048

NC, NS = 2, 16
NW = NC * NS
C = 128
K = 80
EPAD = NW * K * C
FAST_C = 0
KA = 146
KB = 2 * K - KA

NR0 = 10240
RPT0 = NR0 // NS
DUMP0 = N
NR1 = 2176
RPT1 = NR1 // NS
DUMP1 = ROOT


def _make_seg_sum(nr, rpt, with_cnt, filt=False, ka=KA, kb=KB):
    mesh = plsc.VectorSubcoreMesh(
        core_axis_name="c", subcore_axis_name="s",
        num_cores=NC, num_subcores=NS)
    qr = nr // C

    outs = [jax.ShapeDtypeStruct((NC, nr, D), jnp.float32)]
    scratch = [
        pltpu.VMEM((2, C), jnp.int32),
        pltpu.VMEM((2, C), jnp.int32),
        pltpu.VMEM((2, C, D), jnp.float32),
        pltpu.VMEM_SHARED((nr, D), jnp.float32),
        pltpu.SemaphoreType.DMA,
        pltpu.SemaphoreType.DMA,
        pltpu.SemaphoreType.DMA,
        pltpu.SemaphoreType.DMA,
    ]
    if with_cnt:
        outs.append(jax.ShapeDtypeStruct((NC, qr, C), jnp.float32))
        scratch += [
            pltpu.VMEM((qr, C), jnp.float32),
            pltpu.VMEM((qr,), jnp.int32),
            pltpu.VMEM_SHARED((qr, C), jnp.float32),
        ]
    if filt:
        scratch += [
            pltpu.VMEM((ka * C + 2 * C,), jnp.int32),
            pltpu.VMEM((ka * C + 2 * C,), jnp.int32),
            pltpu.VMEM((ka + 1, C), jnp.int32),
        ]

    @functools.partial(
        pl.kernel, out_type=tuple(outs), mesh=mesh,
        scratch_types=scratch,
        compiler_params=pltpu.CompilerParams(needs_layout_passes=False),
    )
    def seg(table, srcs, dsts, zrows, *rest):
        if with_cnt:
            (out, cnt_out, src_v, dst_v, rows_v, accum, semg0, semg1,
             semis, semid, cnt_v, idr_v, cnt_sh) = rest
        elif filt:
            (out, src_v, dst_v, rows_v, accum, semg0, semg1,
             semis, semid, sfil, dfil, dfil2) = rest
        else:
            (out, src_v, dst_v, rows_v, accum, semg0, semg1,
             semis, semid) = rest
        semg = (semg0, semg1)
        c = lax.axis_index("c")
        s = lax.axis_index("s")
        fast = c == FAST_C
        kc = jnp.where(fast, ka, kb)
        ebase = jnp.where(fast, s * ka, NS * ka + s * kb) * C

        def idx_copies(j, b):
            return (
                pltpu.make_async_copy(
                    srcs.at[pl.ds(ebase + j * C, C)], src_v.at[b], semis),
                pltpu.make_async_copy(
                    dsts.at[pl.ds(ebase + j * C, C)], dst_v.at[b], semid),
            )

        def gather_copy(j, b):
            del j
            return pltpu.make_async_copy(
                table.at[src_v.at[b]], rows_v.at[b], semg[b])

        pltpu.sync_copy(zrows, rows_v.at[0])
        base = s * rpt
        for q in range(rpt // C):
            pltpu.sync_copy(rows_v.at[0], accum.at[pl.ds(base + q * C, C)])
        rem = rpt % C
        if rem:
            pltpu.sync_copy(rows_v.at[0, pl.ds(0, rem)],
                            accum.at[pl.ds(base + (rpt // C) * C, rem)])
        if with_cnt:
            pltpu.sync_copy(zrows.at[pl.ds(0, qr)], cnt_v)
            for q in range(qr // 16):
                idr_v[pl.ds(q * 16, 16)] = (
                    lax.iota(jnp.int32, 16) + q * 16)

            @pl.when(s == 0)
            def _():
                pltpu.sync_copy(zrows.at[pl.ds(0, qr)], cnt_sh)

        plsc.subcore_barrier()

        ones = jnp.ones((16,), jnp.float32)

        def count_chunk(b):
            def cstep(i, carry):
                idx = dst_v[b, pl.ds(i * 16, 16)]
                hi = lax.shift_right_logical(idx, 7)
                lo = lax.bitwise_and(idx, 127)
                plsc.addupdate_scatter(cnt_v, [hi, lo], ones)
                return carry
            lax.fori_loop(0, C // 16, cstep, 0)

        if not filt:
            @pl.when(kc > 0)
            def _():
                for d in idx_copies(0, 0):
                    d.start()
                for d in idx_copies(0, 0):
                    d.wait()
                gather_copy(0, 0).start()
                for d in idx_copies(1, 1):
                    d.start()

            def half(j, b):
                nb = 1 - b
                gather_copy(j, b).wait()

                @pl.when(j + 1 < kc)
                def _():
                    for d in idx_copies(j + 1, nb):
                        d.wait()
                    gather_copy(j + 1, nb).start()

                if with_cnt:
                    count_chunk(b)
                pltpu.sync_copy(rows_v.at[b], accum.at[dst_v.at[b]],
                                add=True)

                @pl.when(j + 2 < kc)
                def _():
                    for d in idx_copies(j + 2, b):
                        d.start()

            def round_(r, carry):
                half(2 * r, 0)
                half(2 * r + 1, 1)
                return carry

            lax.fori_loop(0, kc // 2, round_, 0)
        else:
            @pl.when(kc > 0)
            def _():
                for d in idx_copies(0, 0):
                    d.start()

            def fhalf(j, b, n):
                nb = 1 - b
                for d in idx_copies(j, b):
                    d.wait()

                @pl.when(j + 1 < kc)
                def _():
                    for d in idx_copies(j + 1, nb):
                        d.start()

                def cstep(i, n):
                    svec = src_v[b, pl.ds(i * 16, 16)]
                    dvec = dst_v[b, pl.ds(i * 16, 16)]
                    mask = dvec < DUMP1
                    plsc.store_compressed(
                        sfil.at[pl.ds(n, 16)], svec, mask=mask)
                    plsc.store_compressed(
                        dfil.at[pl.ds(n, 16)], dvec, mask=mask)
                    return n + plsc.all_reduce_population_count(mask)[0]

                return lax.fori_loop(0, C // 16, cstep, n)

            def fround(r, n):
                n = fhalf(2 * r, 0, n)
                n = fhalf(2 * r + 1, 1, n)
                return n

            n = lax.fori_loop(0, kc // 2, fround, jnp.int32(0))

            zeros16 = jnp.zeros((16,), jnp.int32)
            dump16 = jnp.full((16,), DUMP1, jnp.int32)
            for t in range(C // 16):
                sfil[pl.ds(n + t * 16, 16)] = zeros16
                dfil[pl.ds(n + t * 16, 16)] = dump16
            nch = (n + C - 1) // C

            def rstep(q, carry):
                def ristep(i, carry2):
                    dfil2[q, pl.ds(i * 16, 16)] = (
                        dfil[pl.ds(q * C + i * 16, 16)])
                    return carry2
                return lax.fori_loop(0, C // 16, ristep, carry)

            lax.fori_loop(0, nch, rstep, 0)

            def gather_f(j, b):
                return pltpu.make_async_copy(
                    table.at[sfil.at[pl.ds(j * C, C)]], rows_v.at[b],
                    semg[b])

            @pl.when(nch > 0)
            def _():
                gather_f(0, 0).start()

            def mhalf(j, b):
                nb = 1 - b
                gather_f(j, b).wait()

                @pl.when(j + 1 < nch)
                def _():
                    gather_f(j + 1, nb).start()

                pltpu.sync_copy(rows_v.at[b], accum.at[dfil2.at[j]],
                                add=True)

            def mround(r, carry):
                mhalf(2 * r, 0)

                @pl.when(2 * r + 1 < nch)
                def _():
                    mhalf(2 * r + 1, 1)
                return carry

            lax.fori_loop(0, (nch + 1) // 2, mround, 0)
        if with_cnt:
            pltpu.sync_copy(cnt_v, cnt_sh.at[idr_v], add=True)
        plsc.subcore_barrier()

        pltpu.sync_copy(accum.at[pl.ds(base, rpt)],
                        out.at[c, pl.ds(base, rpt)])
        if with_cnt:
            @pl.when(s < qr // 8)
            def _():
                pltpu.sync_copy(cnt_sh.at[pl.ds(s * 8, 8)],
                                cnt_out.at[c, pl.ds(s * 8, 8)])

    return seg


_make_seg_sum = functools.lru_cache(maxsize=None)(_make_seg_sum)


def _make_dense(rows, blk, relu):
    nb = rows // blk

    def body(s0, s1, c0, c1, xr, wl, bl, wr, o):
        cnt = jnp.maximum(c0[...] + c1[...], 1.0)
        mean = (s0[...] + s1[...]) / cnt
        h = jnp.dot(mean, wl[...], preferred_element_type=jnp.float32)
        h = h + jnp.dot(xr[...], wr[...], preferred_element_type=jnp.float32)
        h = h + bl[...]
        o[...] = jnp.maximum(h, 0.0) if relu else h

    rb = pl.BlockSpec((blk, D), lambda i: (i, 0))
    cb = pl.BlockSpec((blk, 1), lambda i: (i, 0))
    wb = pl.BlockSpec((D, D), lambda i: (0, 0))
    bb = pl.BlockSpec((1, D), lambda i: (0, 0))
    return pl.pallas_call(
        body,
        grid=(nb,),
        in_specs=[rb, rb, cb, cb, rb, wb, bb, wb],
        out_specs=rb,
        out_shape=jax.ShapeDtypeStruct((rows, D), jnp.float32),
    )


_dense0 = _make_dense(N, 1000, True)
_dense1 = _make_dense(ROOT, 1024, False)


def kernel(x, nbrs, num_root, Wl0, bl0, Wr0, Wl1, bl1, Wr1):
    src = nbrs[0]
    dst = nbrs[1]
    pad = EPAD - E

    srcp = jnp.concatenate([src, jnp.zeros((pad,), jnp.int32)])
    dst0p = jnp.concatenate([dst, jnp.full((pad,), DUMP0, jnp.int32)])

    z = jnp.zeros((C, D), jnp.float32)

    p0, cnt = _make_seg_sum(NR0, RPT0, True)(x, srcp, dst0p, z)
    c0 = cnt[0].reshape(NR0, 1)[:N]
    c1 = cnt[1].reshape(NR0, 1)[:N]
    h = _dense0(p0[0, :N], p0[1, :N], c0, c1,
                x, Wl0.T, bl0.reshape(1, D), Wr0.T)

    (p1,) = _make_seg_sum(NR1, RPT1, False, filt=True,
                          ka=104, kb=56)(h, srcp, dst0p, z)
    out = _dense1(p1[0, :ROOT], p1[1, :ROOT], c0[:ROOT], c1[:ROOT],
                  h[:ROOT], Wl1.T, bl1.reshape(1, D), Wr1.T)
    return out
